# baseline (speedup 1.0000x reference)
import functools

import jax
import jax.numpy as jnp
from jax import lax
from jax.experimental import pallas as pl
from jax.experimental.pallas import tpu as pltpu
from jax.experimental.pallas import tpu_sc as plsc

N = 20000
NPAD = 20480
NW = 32
SRC_PER_W = 2 * NPAD // NW
SBLK = 8
UNROLL = 4
NTV = NPAD // 16
BIG = 1e30


def _sc_chamfer(src_x, src_y, src_z, src_q, src_m, tgt_x, tgt_y, tgt_z, tgt_q):
    mesh = plsc.VectorSubcoreMesh(core_axis_name="c", subcore_axis_name="s")

    @functools.partial(
        pl.kernel,
        mesh=mesh,
        out_type=jax.ShapeDtypeStruct((NW, 16), jnp.float32),
        compiler_params=pltpu.CompilerParams(needs_layout_passes=False),
        scratch_types=[
            pltpu.VMEM((NPAD,), jnp.float32),
            pltpu.VMEM((NPAD,), jnp.float32),
            pltpu.VMEM((NPAD,), jnp.float32),
            pltpu.VMEM((NPAD,), jnp.float32),
            pltpu.VMEM((SRC_PER_W + 16,), jnp.float32),
            pltpu.VMEM((SRC_PER_W + 16,), jnp.float32),
            pltpu.VMEM((SRC_PER_W + 16,), jnp.float32),
            pltpu.VMEM((SRC_PER_W + 16,), jnp.float32),
            pltpu.VMEM((SRC_PER_W + 16,), jnp.float32),
            pltpu.VMEM((16,), jnp.float32),
        ],
    )
    def body(src_x_h, src_y_h, src_z_h, src_q_h, src_m_h,
             tgt_x_h, tgt_y_h, tgt_z_h, tgt_q_h, out_h,
             tx_v, ty_v, tz_v, tq_v, sx_v, sy_v, sz_v, sq_v, sm_v, out_v):
        c = lax.axis_index("c")
        s = lax.axis_index("s")
        wid = c * 16 + s
        sel = wid // 16
        base = wid * SRC_PER_W

        pltpu.sync_copy(tgt_x_h.at[sel], tx_v)
        pltpu.sync_copy(tgt_y_h.at[sel], ty_v)
        pltpu.sync_copy(tgt_z_h.at[sel], tz_v)
        pltpu.sync_copy(tgt_q_h.at[sel], tq_v)
        pltpu.sync_copy(src_x_h.at[pl.ds(base, SRC_PER_W)],
                        sx_v.at[pl.ds(0, SRC_PER_W)])
        pltpu.sync_copy(src_y_h.at[pl.ds(base, SRC_PER_W)],
                        sy_v.at[pl.ds(0, SRC_PER_W)])
        pltpu.sync_copy(src_z_h.at[pl.ds(base, SRC_PER_W)],
                        sz_v.at[pl.ds(0, SRC_PER_W)])
        pltpu.sync_copy(src_q_h.at[pl.ds(base, SRC_PER_W)],
                        sq_v.at[pl.ds(0, SRC_PER_W)])
        pltpu.sync_copy(src_m_h.at[pl.ds(base, SRC_PER_W)],
                        sm_v.at[pl.ds(0, SRC_PER_W)])

        def src_block(b, psum):
            i0 = b * SBLK
            vx = sx_v[pl.ds(i0, 16)]
            vy = sy_v[pl.ds(i0, 16)]
            vz = sz_v[pl.ds(i0, 16)]
            bx, by, bz = [], [], []
            for k in range(SBLK):
                bx.append(jnp.full((16,), vx[k], jnp.float32))
                by.append(jnp.full((16,), vy[k], jnp.float32))
                bz.append(jnp.full((16,), vz[k], jnp.float32))

            init = tuple(jnp.full((16,), 3.0e38, jnp.float32)
                         for _ in range(SBLK))

            def tgt_iter(j, accs):
                accs = list(accs)
                o = j * (16 * UNROLL)
                for u in range(UNROLL):
                    tx = tx_v[pl.ds(o + u * 16, 16)]
                    ty = ty_v[pl.ds(o + u * 16, 16)]
                    tz = tz_v[pl.ds(o + u * 16, 16)]
                    tq = tq_v[pl.ds(o + u * 16, 16)]
                    for k in range(SBLK):
                        d = tq + bx[k] * tx + by[k] * ty + bz[k] * tz
                        accs[k] = jnp.minimum(accs[k], d)
                return tuple(accs)

            accs = lax.fori_loop(0, NTV // UNROLL, tgt_iter, init)

            qv = sq_v[pl.ds(i0, 16)]
            mv = sm_v[pl.ds(i0, 16)]
            for k in range(SBLK):
                psum = psum + (jnp.min(accs[k]) + qv[k]) * mv[k]
            return psum

        psum = lax.fori_loop(0, SRC_PER_W // SBLK, src_block,
                             jnp.float32(0.0))

        lane = lax.broadcasted_iota(jnp.int32, (16,), 0)
        out_v[...] = jnp.where(lane == 0, psum, 0.0)
        pltpu.sync_copy(out_v, out_h.at[wid])

    return body(src_x, src_y, src_z, src_q, src_m, tgt_x, tgt_y, tgt_z, tgt_q)


def kernel(pred, target):
    pad = NPAD - N
    px, py, pz = pred[:, 0], pred[:, 1], pred[:, 2]
    tx, ty, tz = target[:, 0], target[:, 1], target[:, 2]
    qp = px * px + py * py + pz * pz
    qt = tx * tx + ty * ty + tz * tz

    zpad = jnp.zeros((pad,), jnp.float32)
    bpad = jnp.full((pad,), BIG, jnp.float32)
    mask1 = jnp.concatenate([jnp.ones((N,), jnp.float32), zpad])

    src_x = jnp.concatenate([-2.0 * px, zpad, -2.0 * tx, zpad])
    src_y = jnp.concatenate([-2.0 * py, zpad, -2.0 * ty, zpad])
    src_z = jnp.concatenate([-2.0 * pz, zpad, -2.0 * tz, zpad])
    src_q = jnp.concatenate([qp, zpad, qt, zpad])
    src_m = jnp.concatenate([mask1, mask1])

    tgt_x = jnp.stack([jnp.concatenate([tx, zpad]), jnp.concatenate([px, zpad])])
    tgt_y = jnp.stack([jnp.concatenate([ty, zpad]), jnp.concatenate([py, zpad])])
    tgt_z = jnp.stack([jnp.concatenate([tz, zpad]), jnp.concatenate([pz, zpad])])
    tgt_q = jnp.stack([jnp.concatenate([qt, bpad]), jnp.concatenate([qp, bpad])])

    partials = _sc_chamfer(src_x, src_y, src_z, src_q, src_m,
                           tgt_x, tgt_y, tgt_z, tgt_q)
    return partials.sum() / jnp.float32(N)


# speedup vs baseline: 1.8809x; 1.8809x over previous
import functools

import jax
import jax.numpy as jnp
from jax import lax
from jax.experimental import pallas as pl
from jax.experimental.pallas import tpu as pltpu
from jax.experimental.pallas import tpu_sc as plsc

N = 20000
NPAD = 20480
NW = 32
SRC_PER_W = 2 * NPAD // NW
SBLK = 8
UNROLL = 2
NTV = N // 16


def _sc_chamfer(src_x, src_y, src_z, src_q, src_m, tgt_x, tgt_y, tgt_z, tgt_q):
    mesh = plsc.VectorSubcoreMesh(core_axis_name="c", subcore_axis_name="s")

    @functools.partial(
        pl.kernel,
        mesh=mesh,
        out_type=jax.ShapeDtypeStruct((NW, 16), jnp.float32),
        compiler_params=pltpu.CompilerParams(needs_layout_passes=False),
        scratch_types=[
            pltpu.VMEM((N,), jnp.float32),
            pltpu.VMEM((N,), jnp.float32),
            pltpu.VMEM((N,), jnp.float32),
            pltpu.VMEM((N,), jnp.float32),
            pltpu.VMEM((SRC_PER_W + 16,), jnp.float32),
            pltpu.VMEM((SRC_PER_W + 16,), jnp.float32),
            pltpu.VMEM((SRC_PER_W + 16,), jnp.float32),
            pltpu.VMEM((SRC_PER_W + 16,), jnp.float32),
            pltpu.VMEM((SRC_PER_W + 16,), jnp.float32),
            pltpu.VMEM((16,), jnp.float32),
        ],
    )
    def body(src_x_h, src_y_h, src_z_h, src_q_h, src_m_h,
             tgt_x_h, tgt_y_h, tgt_z_h, tgt_q_h, out_h,
             tx_v, ty_v, tz_v, tq_v, sx_v, sy_v, sz_v, sq_v, sm_v, out_v):
        c = lax.axis_index("c")
        s = lax.axis_index("s")
        wid = c * 16 + s
        sel = wid // 16
        base = wid * SRC_PER_W

        pltpu.sync_copy(tgt_x_h.at[sel], tx_v)
        pltpu.sync_copy(tgt_y_h.at[sel], ty_v)
        pltpu.sync_copy(tgt_z_h.at[sel], tz_v)
        pltpu.sync_copy(tgt_q_h.at[sel], tq_v)
        pltpu.sync_copy(src_x_h.at[pl.ds(base, SRC_PER_W)],
                        sx_v.at[pl.ds(0, SRC_PER_W)])
        pltpu.sync_copy(src_y_h.at[pl.ds(base, SRC_PER_W)],
                        sy_v.at[pl.ds(0, SRC_PER_W)])
        pltpu.sync_copy(src_z_h.at[pl.ds(base, SRC_PER_W)],
                        sz_v.at[pl.ds(0, SRC_PER_W)])
        pltpu.sync_copy(src_q_h.at[pl.ds(base, SRC_PER_W)],
                        sq_v.at[pl.ds(0, SRC_PER_W)])
        pltpu.sync_copy(src_m_h.at[pl.ds(base, SRC_PER_W)],
                        sm_v.at[pl.ds(0, SRC_PER_W)])

        def src_block(b, psum):
            i0 = b * SBLK
            vx = sx_v[pl.ds(i0, 16)]
            vy = sy_v[pl.ds(i0, 16)]
            vz = sz_v[pl.ds(i0, 16)]
            bx, by, bz = [], [], []
            for k in range(SBLK):
                bx.append(jnp.full((16,), vx[k], jnp.float32))
                by.append(jnp.full((16,), vy[k], jnp.float32))
                bz.append(jnp.full((16,), vz[k], jnp.float32))

            init = tuple(jnp.full((16,), 3.0e38, jnp.float32)
                         for _ in range(SBLK))

            def tgt_iter(j, accs):
                accs = list(accs)
                o = j * (16 * UNROLL)
                for u in range(UNROLL):
                    tx = tx_v[pl.ds(o + u * 16, 16)]
                    ty = ty_v[pl.ds(o + u * 16, 16)]
                    tz = tz_v[pl.ds(o + u * 16, 16)]
                    tq = tq_v[pl.ds(o + u * 16, 16)]
                    for k in range(SBLK):
                        d = tq + bx[k] * tx + by[k] * ty + bz[k] * tz
                        accs[k] = jnp.minimum(accs[k], d)
                return tuple(accs)

            accs = lax.fori_loop(0, NTV // UNROLL, tgt_iter, init)

            qv = sq_v[pl.ds(i0, 16)]
            mv = sm_v[pl.ds(i0, 16)]
            for k in range(SBLK):
                psum = psum + (jnp.min(accs[k]) + qv[k]) * mv[k]
            return psum

        psum = lax.fori_loop(0, SRC_PER_W // SBLK, src_block,
                             jnp.float32(0.0))

        lane = lax.broadcasted_iota(jnp.int32, (16,), 0)
        out_v[...] = jnp.where(lane == 0, psum, 0.0)
        pltpu.sync_copy(out_v, out_h.at[wid])

    return body(src_x, src_y, src_z, src_q, src_m, tgt_x, tgt_y, tgt_z, tgt_q)


def kernel(pred, target):
    pad = NPAD - N
    px, py, pz = pred[:, 0], pred[:, 1], pred[:, 2]
    tx, ty, tz = target[:, 0], target[:, 1], target[:, 2]
    qp = px * px + py * py + pz * pz
    qt = tx * tx + ty * ty + tz * tz

    zpad = jnp.zeros((pad,), jnp.float32)
    mask1 = jnp.concatenate([jnp.ones((N,), jnp.float32), zpad])

    src_x = jnp.concatenate([-2.0 * px, zpad, -2.0 * tx, zpad])
    src_y = jnp.concatenate([-2.0 * py, zpad, -2.0 * ty, zpad])
    src_z = jnp.concatenate([-2.0 * pz, zpad, -2.0 * tz, zpad])
    src_q = jnp.concatenate([qp, zpad, qt, zpad])
    src_m = jnp.concatenate([mask1, mask1])

    tgt_x = jnp.stack([tx, px])
    tgt_y = jnp.stack([ty, py])
    tgt_z = jnp.stack([tz, pz])
    tgt_q = jnp.stack([qt, qp])

    partials = _sc_chamfer(src_x, src_y, src_z, src_q, src_m,
                           tgt_x, tgt_y, tgt_z, tgt_q)
    return partials.sum() / jnp.float32(N)


# speedup vs baseline: 2.6288x; 1.3976x over previous
import functools

import jax
import jax.numpy as jnp
from jax import lax
from jax.experimental import pallas as pl
from jax.experimental.pallas import tpu as pltpu
from jax.experimental.pallas import tpu_sc as plsc

N = 20000
NPAD = 20480
NW = 32
SRC_PER_W = NPAD // NW
SBLK = 8
UNROLL = 2
NTV = N // 16
CB_T = 640
NCOL = NPAD


def _sc_sweep(src_x, src_y, src_z, src_q, src_m, tgt_x, tgt_y, tgt_z, tgt_q):
    mesh = plsc.VectorSubcoreMesh(core_axis_name="c", subcore_axis_name="s")

    @functools.partial(
        pl.kernel,
        mesh=mesh,
        out_type=(jax.ShapeDtypeStruct((NW, 16), jnp.float32),
                  jax.ShapeDtypeStruct((NW, NCOL), jnp.float32)),
        compiler_params=pltpu.CompilerParams(needs_layout_passes=False),
        scratch_types=[
            pltpu.VMEM((N,), jnp.float32),
            pltpu.VMEM((N,), jnp.float32),
            pltpu.VMEM((N,), jnp.float32),
            pltpu.VMEM((N,), jnp.float32),
            pltpu.VMEM((NCOL,), jnp.float32),
            pltpu.VMEM((SRC_PER_W + 16,), jnp.float32),
            pltpu.VMEM((SRC_PER_W + 16,), jnp.float32),
            pltpu.VMEM((SRC_PER_W + 16,), jnp.float32),
            pltpu.VMEM((SRC_PER_W + 16,), jnp.float32),
            pltpu.VMEM((SRC_PER_W + 16,), jnp.float32),
            pltpu.VMEM((16,), jnp.float32),
        ],
    )
    def body(src_x_h, src_y_h, src_z_h, src_q_h, src_m_h,
             tgt_x_h, tgt_y_h, tgt_z_h, tgt_q_h,
             rows_h, cols_h,
             tx_v, ty_v, tz_v, tq_v, col_v,
             sx_v, sy_v, sz_v, sq_v, sm_v, out_v):
        c = lax.axis_index("c")
        s = lax.axis_index("s")
        wid = c * 16 + s
        base = wid * SRC_PER_W

        pltpu.sync_copy(tgt_x_h, tx_v)
        pltpu.sync_copy(tgt_y_h, ty_v)
        pltpu.sync_copy(tgt_z_h, tz_v)
        pltpu.sync_copy(tgt_q_h, tq_v)
        pltpu.sync_copy(src_x_h.at[pl.ds(base, SRC_PER_W)],
                        sx_v.at[pl.ds(0, SRC_PER_W)])
        pltpu.sync_copy(src_y_h.at[pl.ds(base, SRC_PER_W)],
                        sy_v.at[pl.ds(0, SRC_PER_W)])
        pltpu.sync_copy(src_z_h.at[pl.ds(base, SRC_PER_W)],
                        sz_v.at[pl.ds(0, SRC_PER_W)])
        pltpu.sync_copy(src_q_h.at[pl.ds(base, SRC_PER_W)],
                        sq_v.at[pl.ds(0, SRC_PER_W)])
        pltpu.sync_copy(src_m_h.at[pl.ds(base, SRC_PER_W)],
                        sm_v.at[pl.ds(0, SRC_PER_W)])

        big = jnp.full((16,), 3.0e38, jnp.float32)

        def col_init(j, carry):
            col_v[pl.ds(j * 16, 16)] = big
            return carry

        lax.fori_loop(0, NTV, col_init, jnp.float32(0.0))

        zeros16 = jnp.zeros((16,), jnp.float32)

        def col_tail_init(j, carry):
            col_v[pl.ds(NTV * 16 + j * 16, 16)] = zeros16
            return carry

        lax.fori_loop(0, (NCOL - N) // 16, col_tail_init, jnp.float32(0.0))

        def src_block(b, psum):
            i0 = b * SBLK
            vx = sx_v[pl.ds(i0, 16)]
            vy = sy_v[pl.ds(i0, 16)]
            vz = sz_v[pl.ds(i0, 16)]
            vq = sq_v[pl.ds(i0, 16)]
            bx, by, bz, bq = [], [], [], []
            for k in range(SBLK):
                bx.append(jnp.full((16,), vx[k], jnp.float32))
                by.append(jnp.full((16,), vy[k], jnp.float32))
                bz.append(jnp.full((16,), vz[k], jnp.float32))
                bq.append(jnp.full((16,), vq[k], jnp.float32))

            init = tuple(big for _ in range(SBLK))

            def tgt_iter(j, accs):
                accs = list(accs)
                o = j * (16 * UNROLL)
                for u in range(UNROLL):
                    off = o + u * 16
                    tx = tx_v[pl.ds(off, 16)]
                    ty = ty_v[pl.ds(off, 16)]
                    tz = tz_v[pl.ds(off, 16)]
                    tq = tq_v[pl.ds(off, 16)]
                    cmin = None
                    for k in range(SBLK):
                        e = tq + bx[k] * tx + by[k] * ty + bz[k] * tz
                        accs[k] = jnp.minimum(accs[k], e)
                        d = e + bq[k]
                        cmin = d if cmin is None else jnp.minimum(cmin, d)
                    col_v[pl.ds(off, 16)] = jnp.minimum(
                        col_v[pl.ds(off, 16)], cmin)
                return tuple(accs)

            accs = lax.fori_loop(0, NTV // UNROLL, tgt_iter, init)

            mv = sm_v[pl.ds(i0, 16)]
            for k in range(SBLK):
                psum = psum + (jnp.min(accs[k]) + vq[k]) * mv[k]
            return psum

        psum = lax.fori_loop(0, SRC_PER_W // SBLK, src_block,
                             jnp.float32(0.0))

        lane = lax.broadcasted_iota(jnp.int32, (16,), 0)
        out_v[...] = jnp.where(lane == 0, psum, 0.0)
        pltpu.sync_copy(out_v, rows_h.at[wid])
        pltpu.sync_copy(col_v, cols_h.at[wid])

    return body(src_x, src_y, src_z, src_q, src_m,
                tgt_x, tgt_y, tgt_z, tgt_q)


def _sc_combine(cols):
    mesh = plsc.VectorSubcoreMesh(core_axis_name="c", subcore_axis_name="s")

    @functools.partial(
        pl.kernel,
        mesh=mesh,
        out_type=jax.ShapeDtypeStruct((NW, 16), jnp.float32),
        compiler_params=pltpu.CompilerParams(needs_layout_passes=False),
        scratch_types=[
            pltpu.VMEM((NW, CB_T), jnp.float32),
            pltpu.VMEM((16,), jnp.float32),
        ],
    )
    def body(cols_h, out_h, rows_v, out_v):
        c = lax.axis_index("c")
        s = lax.axis_index("s")
        wid = c * 16 + s
        base = wid * CB_T

        pltpu.sync_copy(cols_h.at[:, pl.ds(base, CB_T)], rows_v)

        def it(jv, vsum):
            o = jv * 16
            m = rows_v[0, pl.ds(o, 16)]
            for r in range(1, NW):
                m = jnp.minimum(m, rows_v[r, pl.ds(o, 16)])
            return vsum + m

        vsum = lax.fori_loop(0, CB_T // 16, it,
                             jnp.zeros((16,), jnp.float32))
        total = jnp.sum(vsum)

        lane = lax.broadcasted_iota(jnp.int32, (16,), 0)
        out_v[...] = jnp.where(lane == 0, total, 0.0)
        pltpu.sync_copy(out_v, out_h.at[wid])

    return body(cols)


def kernel(pred, target):
    pad = NPAD - N
    px, py, pz = pred[:, 0], pred[:, 1], pred[:, 2]
    tx, ty, tz = target[:, 0], target[:, 1], target[:, 2]
    qp = px * px + py * py + pz * pz
    qt = tx * tx + ty * ty + tz * tz

    zpad = jnp.zeros((pad,), jnp.float32)
    bpad = jnp.full((pad,), 1e30, jnp.float32)

    src_x = jnp.concatenate([-2.0 * px, zpad])
    src_y = jnp.concatenate([-2.0 * py, zpad])
    src_z = jnp.concatenate([-2.0 * pz, zpad])
    src_q = jnp.concatenate([qp, bpad])
    src_m = jnp.concatenate([jnp.ones((N,), jnp.float32), zpad])

    rows, cols = _sc_sweep(src_x, src_y, src_z, src_q, src_m,
                           tx, ty, tz, qt)
    colsums = _sc_combine(cols)
    return (rows.sum() + colsums.sum()) / jnp.float32(N)
